# baseline (speedup 1.0000x reference)
import functools

import jax
import jax.numpy as jnp
from jax import lax
from jax.experimental import pallas as pl
from jax.experimental.pallas import tpu as pltpu
from jax.experimental.pallas import tpu_sc as plsc

FEATURES = 32
BATCH = 4096
HIST = 50
NUM_WORKERS = 32
BBLK = BATCH // NUM_WORKERS
NROWS = 1000000
QROWS = NROWS // 4
L = 16
NBUF = 2
NTILE = NROWS // 128
TAIL = NROWS - NTILE * 128
TPW = NTILE // NUM_WORKERS

_mesh = plsc.VectorSubcoreMesh(core_axis_name="c", subcore_axis_name="s")


def _build_repack():
  @functools.partial(
      pl.kernel,
      mesh=_mesh,
      compiler_params=pltpu.CompilerParams(needs_layout_passes=False),
      out_type=jax.ShapeDtypeStruct((NROWS * FEATURES,), jnp.float32),
      scratch_types=[
          pltpu.VMEM((FEATURES, 128), jnp.float32),
          pltpu.VMEM((4096,), jnp.float32),
      ],
  )
  def k(tab_hbm, tail_hbm, out_hbm, vbuf, obuf):
    wid = lax.axis_index("s") * 2 + lax.axis_index("c")
    iota = lax.iota(jnp.int32, L)
    base_g = [g * 512 + iota * 32 for g in range(8)]

    def transpose_block():
      def dbody(d, carry):
        colbase = lax.bitwise_and(iota + d, jnp.full((L,), L - 1, jnp.int32))
        for c0 in range(0, FEATURES, L):
          cvec = colbase + c0
          for g in range(8):
            ilvec = g * L + iota
            v = plsc.load_gather(vbuf, [cvec, ilvec])
            plsc.store_scatter(obuf, [base_g[g] + cvec], v)
        return carry

      lax.fori_loop(0, L, dbody, jnp.int32(0))

    def do_tile(tc):
      pltpu.sync_copy(tab_hbm.at[:, pl.ds(tc * 128, 128)], vbuf)
      transpose_block()
      pltpu.sync_copy(obuf, out_hbm.at[pl.ds(tc * 4096, 4096)])

    def body(t, carry):
      do_tile(t * NUM_WORKERS + wid)
      return carry

    lax.fori_loop(0, TPW, body, jnp.int32(0))
    @pl.when(wid < 4)
    def _():
      do_tile(jnp.int32(NTILE - 4) + wid)

    @pl.when(wid == 4)
    def _():
      pltpu.sync_copy(tail_hbm, vbuf)
      transpose_block()
      pltpu.sync_copy(obuf.at[pl.ds(0, TAIL * FEATURES)],
                      out_hbm.at[pl.ds(NTILE * 4096, TAIL * FEATURES)])

  return k


def _build_gather():
  @functools.partial(
      pl.kernel,
      mesh=_mesh,
      compiler_params=pltpu.CompilerParams(
          use_tc_tiling_on_sc=False, needs_layout_passes=False),
      out_type=jax.ShapeDtypeStruct((HIST, FEATURES, BATCH), jnp.float32),
      scratch_types=[
          pltpu.VMEM((HIST, BBLK), jnp.int32),
          [pltpu.VMEM((BBLK,), jnp.int32) for _ in range(NBUF)],
          [pltpu.VMEM((BBLK,), jnp.int32) for _ in range(NBUF)],
          [pltpu.VMEM((BBLK, 128), jnp.float32) for _ in range(NBUF)],
          [pltpu.VMEM((FEATURES, BBLK), jnp.float32) for _ in range(NBUF)],
          [pltpu.SemaphoreType.DMA for _ in range(NBUF)],
      ],
  )
  def k(idx_hbm, table_hbm, out_hbm, idx_v, rowid_v, qoff_v, buf_v, obuf_v,
        gsem):
    wid = lax.axis_index("s") * 2 + lax.axis_index("c")
    b0 = wid * BBLK
    pltpu.sync_copy(idx_hbm.at[:, pl.ds(b0, BBLK)], idx_v)

    iota = lax.iota(jnp.int32, L)

    def prep(h, slot):
      for g in range(BBLK // L):
        v = idx_v[h, pl.ds(g * L, L)]
        rowid_v[slot][pl.ds(g * L, L)] = v >> 2
        qoff_v[slot][pl.ds(g * L, L)] = (v & 3) << 5
      pltpu.async_copy(table_hbm.at[rowid_v[slot]], buf_v[slot], gsem[slot])

    def step(h, slot):
      pltpu.make_async_copy(
          table_hbm.at[rowid_v[slot]], buf_v[slot], gsem[slot]).wait()

      def dbody(d, carry):
        colbase = lax.bitwise_and(iota + d, jnp.full((L,), L - 1, jnp.int32))
        for g in range(BBLK // L):
          rows = g * L + iota
          qoff = qoff_v[slot][pl.ds(g * L, L)]
          for c0 in range(0, FEATURES, L):
            cols = colbase + c0
            v = plsc.load_gather(buf_v[slot], [rows, qoff + cols])
            plsc.store_scatter(obuf_v[slot], [cols, rows], v)
        return carry

      lax.fori_loop(0, L, dbody, jnp.int32(0))
      pltpu.sync_copy(obuf_v[slot], out_hbm.at[h, :, pl.ds(b0, BBLK)])
      prep(jnp.minimum(h + NBUF, HIST - 1), slot)

    for s in range(NBUF):
      prep(jnp.int32(s), s)

    def body(j, carry):
      for s in range(NBUF):
        step(j * NBUF + s, s)
      return carry

    lax.fori_loop(0, HIST // NBUF, body, jnp.int32(0))

    for s in range(NBUF):
      pltpu.make_async_copy(
          table_hbm.at[rowid_v[s]], buf_v[s], gsem[s]).wait()

  return k


_repack_kernel = _build_repack()
_gather_kernel = _build_gather()


def kernel(inputs, embedding):
  idx_t = inputs.T.astype(jnp.int32)
  tail = jnp.pad(embedding[NTILE * 128:].T, ((0, 0), (0, 128 - TAIL)))
  packed = _repack_kernel(embedding.T, tail)
  table128 = packed.reshape(QROWS, 128)
  out_t = _gather_kernel(idx_t, table128)
  return out_t.transpose(2, 0, 1)


# speedup vs baseline: 1.6760x; 1.6760x over previous
import functools

import jax
import jax.numpy as jnp
from jax import lax
from jax.experimental import pallas as pl
from jax.experimental.pallas import tpu as pltpu
from jax.experimental.pallas import tpu_sc as plsc

FEATURES = 32
BATCH = 4096
HIST = 50
NUM_WORKERS = 32
BBLK = BATCH // NUM_WORKERS
NROWS = 1000000
QROWS = NROWS // 4
L = 16
NBUF = 2
NTILE = NROWS // 128
TAIL = NROWS - NTILE * 128
TPW = NTILE // NUM_WORKERS

_mesh = plsc.VectorSubcoreMesh(core_axis_name="c", subcore_axis_name="s")


def _build_repack():
  @functools.partial(
      pl.kernel,
      mesh=_mesh,
      compiler_params=pltpu.CompilerParams(needs_layout_passes=False),
      out_type=jax.ShapeDtypeStruct((NROWS * FEATURES,), jnp.float32),
      scratch_types=[
          [pltpu.VMEM((FEATURES, 128), jnp.float32) for _ in range(NBUF)],
          [pltpu.VMEM((4096,), jnp.float32) for _ in range(NBUF)],
          [pltpu.SemaphoreType.DMA for _ in range(NBUF)],
          [pltpu.SemaphoreType.DMA for _ in range(NBUF)],
      ],
  )
  def k(tab_hbm, tail_hbm, out_hbm, vbuf, obuf, isem, osem):
    wid = lax.axis_index("s") * 2 + lax.axis_index("c")
    iota = lax.iota(jnp.int32, L)
    base_g = [g * 512 + iota * 32 for g in range(8)]

    def wtile(t):
      return jnp.where(t < TPW, t * NUM_WORKERS + wid, jnp.int32(NTILE - 4) + wid)

    def stage(t, slot):
      pltpu.async_copy(tab_hbm.at[:, pl.ds(wtile(t) * 128, 128)], vbuf[slot],
                       isem[slot])

    def transpose_block(slot):
      def dbody(d4, carry):
        for dd in range(4):
          colbase = lax.bitwise_and(iota + (d4 * 4 + dd),
                                    jnp.full((L,), L - 1, jnp.int32))
          for c0 in range(0, FEATURES, L):
            cvec = colbase + c0
            for g in range(8):
              ilvec = g * L + iota
              v = plsc.load_gather(vbuf[slot], [cvec, ilvec])
              plsc.store_scatter(obuf[slot], [base_g[g] + cvec], v)
        return carry

      lax.fori_loop(0, 4, dbody, jnp.int32(0))

    def work(t, slot):
      pltpu.make_async_copy(tab_hbm.at[:, pl.ds(wtile(t) * 128, 128)],
                            vbuf[slot], isem[slot]).wait()
      transpose_block(slot)
      pltpu.async_copy(obuf[slot], out_hbm.at[pl.ds(wtile(t) * 4096, 4096)],
                       osem[slot])

    nt = TPW + jnp.where(wid < 4, 1, 0)

    for s in range(NBUF):
      stage(jnp.int32(s), s)

    def body(j, carry):
      for s in range(NBUF):
        t = j * NBUF + s

        @pl.when(t < nt)
        def _():
          @pl.when(t >= NBUF)
          def _():
            pltpu.make_async_copy(
                obuf[s], out_hbm.at[pl.ds(wtile(t) * 4096, 4096)],
                osem[s]).wait()
          work(t, s)

          @pl.when(t + NBUF < nt)
          def _():
            stage(t + NBUF, s)
      return carry

    lax.fori_loop(0, (TPW + 1 + NBUF - 1) // NBUF, body, jnp.int32(0))
    for s in range(NBUF):
      @pl.when(jnp.minimum(nt, jnp.int32(NBUF)) > s)
      def _():
        pltpu.make_async_copy(obuf[s], out_hbm.at[pl.ds(0, 4096)],
                              osem[s]).wait()

    @pl.when(wid == 4)
    def _():
      pltpu.sync_copy(tail_hbm, vbuf[0])
      transpose_block(0)
      pltpu.sync_copy(obuf[0].at[pl.ds(0, TAIL * FEATURES)],
                      out_hbm.at[pl.ds(NTILE * 4096, TAIL * FEATURES)])

  return k


def _build_gather():
  @functools.partial(
      pl.kernel,
      mesh=_mesh,
      compiler_params=pltpu.CompilerParams(
          use_tc_tiling_on_sc=False, needs_layout_passes=False),
      out_type=jax.ShapeDtypeStruct((HIST, FEATURES, BATCH), jnp.float32),
      scratch_types=[
          pltpu.VMEM((HIST, BBLK), jnp.int32),
          [pltpu.VMEM((BBLK,), jnp.int32) for _ in range(NBUF)],
          [pltpu.VMEM((BBLK,), jnp.int32) for _ in range(NBUF)],
          [pltpu.VMEM((BBLK, 128), jnp.float32) for _ in range(NBUF)],
          [pltpu.VMEM((FEATURES, BBLK), jnp.float32) for _ in range(NBUF)],
          [pltpu.SemaphoreType.DMA for _ in range(NBUF)],
      ],
  )
  def k(idx_hbm, table_hbm, out_hbm, idx_v, rowid_v, qoff_v, buf_v, obuf_v,
        gsem):
    wid = lax.axis_index("s") * 2 + lax.axis_index("c")
    b0 = wid * BBLK
    pltpu.sync_copy(idx_hbm.at[:, pl.ds(b0, BBLK)], idx_v)

    iota = lax.iota(jnp.int32, L)

    def prep(h, slot):
      for g in range(BBLK // L):
        v = idx_v[h, pl.ds(g * L, L)]
        rowid_v[slot][pl.ds(g * L, L)] = v >> 2
        qoff_v[slot][pl.ds(g * L, L)] = (v & 3) << 5
      pltpu.async_copy(table_hbm.at[rowid_v[slot]], buf_v[slot], gsem[slot])

    def step(h, slot):
      pltpu.make_async_copy(
          table_hbm.at[rowid_v[slot]], buf_v[slot], gsem[slot]).wait()

      def dbody(d, carry):
        colbase = lax.bitwise_and(iota + d, jnp.full((L,), L - 1, jnp.int32))
        for g in range(BBLK // L):
          rows = g * L + iota
          qoff = qoff_v[slot][pl.ds(g * L, L)]
          for c0 in range(0, FEATURES, L):
            cols = colbase + c0
            v = plsc.load_gather(buf_v[slot], [rows, qoff + cols])
            plsc.store_scatter(obuf_v[slot], [cols, rows], v)
        return carry

      lax.fori_loop(0, L, dbody, jnp.int32(0))
      pltpu.sync_copy(obuf_v[slot], out_hbm.at[h, :, pl.ds(b0, BBLK)])
      prep(jnp.minimum(h + NBUF, HIST - 1), slot)

    for s in range(NBUF):
      prep(jnp.int32(s), s)

    def body(j, carry):
      for s in range(NBUF):
        step(j * NBUF + s, s)
      return carry

    lax.fori_loop(0, HIST // NBUF, body, jnp.int32(0))

    for s in range(NBUF):
      pltpu.make_async_copy(
          table_hbm.at[rowid_v[s]], buf_v[s], gsem[s]).wait()

  return k


_repack_kernel = _build_repack()
_gather_kernel = _build_gather()


def kernel(inputs, embedding):
  idx_t = inputs.T.astype(jnp.int32)
  tail = jnp.pad(embedding[NTILE * 128:].T, ((0, 0), (0, 128 - TAIL)))
  packed = _repack_kernel(embedding.T, tail)
  table128 = packed.reshape(QROWS, 128)
  out_t = _gather_kernel(idx_t, table128)
  return out_t.transpose(2, 0, 1)


# speedup vs baseline: 1.8212x; 1.0867x over previous
import functools

import jax
import jax.numpy as jnp
from jax import lax
from jax.experimental import pallas as pl
from jax.experimental.pallas import tpu as pltpu
from jax.experimental.pallas import tpu_sc as plsc

FEATURES = 32
BATCH = 4096
HIST = 50
NUM_WORKERS = 32
BBLK = BATCH // NUM_WORKERS
NROWS = 1000000
QROWS = NROWS // 4
L = 16
NBUF = 2
NTILE = NROWS // 128
TAIL = NROWS - NTILE * 128
TPW = NTILE // NUM_WORKERS
RBUF = 3

_mesh = plsc.VectorSubcoreMesh(core_axis_name="c", subcore_axis_name="s")


def _build_repack():
  @functools.partial(
      pl.kernel,
      mesh=_mesh,
      compiler_params=pltpu.CompilerParams(needs_layout_passes=False),
      out_type=jax.ShapeDtypeStruct((NROWS * FEATURES,), jnp.float32),
      scratch_types=[
          [pltpu.VMEM((FEATURES, 128), jnp.float32) for _ in range(RBUF)],
          [pltpu.VMEM((4096,), jnp.float32) for _ in range(RBUF)],
          [pltpu.SemaphoreType.DMA for _ in range(RBUF)],
          [pltpu.SemaphoreType.DMA for _ in range(RBUF)],
      ],
  )
  def k(tab_hbm, tail_hbm, out_hbm, vbuf, obuf, isem, osem):
    wid = lax.axis_index("s") * 2 + lax.axis_index("c")
    iota = lax.iota(jnp.int32, L)
    base_g = [g * 512 + iota * 32 for g in range(8)]

    def wtile(t):
      return jnp.where(t < TPW, t * NUM_WORKERS + wid, jnp.int32(NTILE - 4) + wid)

    def stage(t, slot):
      pltpu.async_copy(tab_hbm.at[:, pl.ds(wtile(t) * 128, 128)], vbuf[slot],
                       isem[slot])

    def transpose_block(slot):
      def dbody(d8, carry):
        for dd in range(8):
          colbase = lax.bitwise_and(iota + (d8 * 8 + dd),
                                    jnp.full((L,), L - 1, jnp.int32))
          for c0 in range(0, FEATURES, L):
            cvec = colbase + c0
            for g in range(8):
              ilvec = g * L + iota
              v = plsc.load_gather(vbuf[slot], [cvec, ilvec])
              plsc.store_scatter(obuf[slot], [base_g[g] + cvec], v)
        return carry

      lax.fori_loop(0, 2, dbody, jnp.int32(0))

    def work(t, slot):
      pltpu.make_async_copy(tab_hbm.at[:, pl.ds(wtile(t) * 128, 128)],
                            vbuf[slot], isem[slot]).wait()
      transpose_block(slot)
      pltpu.async_copy(obuf[slot], out_hbm.at[pl.ds(wtile(t) * 4096, 4096)],
                       osem[slot])

    nt = TPW + jnp.where(wid < 4, 1, 0)

    for s in range(RBUF):
      stage(jnp.int32(s), s)

    def body(j, carry):
      for s in range(RBUF):
        t = j * RBUF + s

        @pl.when(t < nt)
        def _():
          @pl.when(t >= RBUF)
          def _():
            pltpu.make_async_copy(
                obuf[s], out_hbm.at[pl.ds(wtile(t) * 4096, 4096)],
                osem[s]).wait()
          work(t, s)

          @pl.when(t + RBUF < nt)
          def _():
            stage(t + RBUF, s)
      return carry

    lax.fori_loop(0, (TPW + 1 + RBUF - 1) // RBUF, body, jnp.int32(0))
    for s in range(RBUF):
      @pl.when(jnp.minimum(nt, jnp.int32(RBUF)) > s)
      def _():
        pltpu.make_async_copy(obuf[s], out_hbm.at[pl.ds(0, 4096)],
                              osem[s]).wait()

    @pl.when(wid == 4)
    def _():
      pltpu.sync_copy(tail_hbm, vbuf[0])
      transpose_block(0)
      pltpu.sync_copy(obuf[0].at[pl.ds(0, TAIL * FEATURES)],
                      out_hbm.at[pl.ds(NTILE * 4096, TAIL * FEATURES)])

  return k


def _build_gather():
  @functools.partial(
      pl.kernel,
      mesh=_mesh,
      compiler_params=pltpu.CompilerParams(
          use_tc_tiling_on_sc=False, needs_layout_passes=False),
      out_type=jax.ShapeDtypeStruct((HIST, FEATURES, BATCH), jnp.float32),
      scratch_types=[
          pltpu.VMEM((HIST, BBLK), jnp.int32),
          [pltpu.VMEM((BBLK,), jnp.int32) for _ in range(NBUF)],
          [pltpu.VMEM((BBLK,), jnp.int32) for _ in range(NBUF)],
          [pltpu.VMEM((BBLK, 128), jnp.float32) for _ in range(NBUF)],
          [pltpu.VMEM((FEATURES, BBLK), jnp.float32) for _ in range(NBUF)],
          [pltpu.SemaphoreType.DMA for _ in range(NBUF)],
          [pltpu.SemaphoreType.DMA for _ in range(NBUF)],
      ],
  )
  def k(idx_hbm, table_hbm, out_hbm, idx_v, rowid_v, qoff_v, buf_v, obuf_v,
        gsem, osem):
    wid = lax.axis_index("s") * 2 + lax.axis_index("c")
    b0 = wid * BBLK
    pltpu.sync_copy(idx_hbm.at[:, pl.ds(b0, BBLK)], idx_v)

    iota = lax.iota(jnp.int32, L)

    def prep(h, slot):
      for g in range(BBLK // L):
        v = idx_v[h, pl.ds(g * L, L)]
        rowid_v[slot][pl.ds(g * L, L)] = v >> 2
        qoff_v[slot][pl.ds(g * L, L)] = (v & 3) << 5
      pltpu.async_copy(table_hbm.at[rowid_v[slot]], buf_v[slot], gsem[slot])

    def step(h, slot):
      pltpu.make_async_copy(
          table_hbm.at[rowid_v[slot]], buf_v[slot], gsem[slot]).wait()

      def dbody(d, carry):
        colbase = lax.bitwise_and(iota + d, jnp.full((L,), L - 1, jnp.int32))
        for g in range(BBLK // L):
          rows = g * L + iota
          qoff = qoff_v[slot][pl.ds(g * L, L)]
          for c0 in range(0, FEATURES, L):
            cols = colbase + c0
            v = plsc.load_gather(buf_v[slot], [rows, qoff + cols])
            plsc.store_scatter(obuf_v[slot], [cols, rows], v)
        return carry

      lax.fori_loop(0, L, dbody, jnp.int32(0))
      pltpu.async_copy(obuf_v[slot], out_hbm.at[h, :, pl.ds(b0, BBLK)],
                       osem[slot])
      prep(jnp.minimum(h + NBUF, HIST - 1), slot)

    for s in range(NBUF):
      prep(jnp.int32(s), s)

    def body(j, carry):
      for s in range(NBUF):
        h = j * NBUF + s

        @pl.when(h >= NBUF)
        def _():
          pltpu.make_async_copy(
              obuf_v[s], out_hbm.at[h, :, pl.ds(b0, BBLK)], osem[s]).wait()
        step(h, s)
      return carry

    lax.fori_loop(0, HIST // NBUF, body, jnp.int32(0))

    for s in range(NBUF):
      pltpu.make_async_copy(
          table_hbm.at[rowid_v[s]], buf_v[s], gsem[s]).wait()
      pltpu.make_async_copy(
          obuf_v[s], out_hbm.at[jnp.int32(0), :, pl.ds(b0, BBLK)],
          osem[s]).wait()

  return k


_repack_kernel = _build_repack()
_gather_kernel = _build_gather()


def kernel(inputs, embedding):
  idx_t = inputs.T.astype(jnp.int32)
  tail = jnp.pad(embedding[NTILE * 128:].T, ((0, 0), (0, 128 - TAIL)))
  packed = _repack_kernel(embedding.T, tail)
  table128 = packed.reshape(QROWS, 128)
  out_t = _gather_kernel(idx_t, table128)
  return out_t.transpose(2, 0, 1)


# speedup vs baseline: 1.9736x; 1.0837x over previous
import functools

import jax
import jax.numpy as jnp
from jax import lax
from jax.experimental import pallas as pl
from jax.experimental.pallas import tpu as pltpu
from jax.experimental.pallas import tpu_sc as plsc

FEATURES = 32
BATCH = 4096
HIST = 50
NUM_WORKERS = 32
BBLK = BATCH // NUM_WORKERS
NROWS = 1000000
QROWS = NROWS // 4
L = 16
NBUF = 2
NTILE = NROWS // 128
TAIL = NROWS - NTILE * 128
TPW = NTILE // NUM_WORKERS
RBUF = 3

_mesh = plsc.VectorSubcoreMesh(core_axis_name="c", subcore_axis_name="s")


def _build_repack():
  @functools.partial(
      pl.kernel,
      mesh=_mesh,
      compiler_params=pltpu.CompilerParams(needs_layout_passes=False),
      out_type=jax.ShapeDtypeStruct((NROWS * FEATURES,), jnp.float32),
      scratch_types=[
          [pltpu.VMEM((FEATURES, 128), jnp.float32) for _ in range(RBUF)],
          [pltpu.VMEM((4096,), jnp.float32) for _ in range(RBUF)],
          [pltpu.SemaphoreType.DMA for _ in range(RBUF)],
          [pltpu.SemaphoreType.DMA for _ in range(RBUF)],
      ],
  )
  def k(tab_hbm, tail_hbm, out_hbm, vbuf, obuf, isem, osem):
    wid = lax.axis_index("s") * 2 + lax.axis_index("c")
    iota = lax.iota(jnp.int32, L)
    base_g = [g * 512 + iota * 32 for g in range(8)]

    def wtile(t):
      return jnp.where(t < TPW, t * NUM_WORKERS + wid, jnp.int32(NTILE - 4) + wid)

    def stage(t, slot):
      pltpu.async_copy(tab_hbm.at[:, pl.ds(wtile(t) * 128, 128)], vbuf[slot],
                       isem[slot])

    def transpose_block(slot):
      def dbody(d8, carry):
        for dd in range(8):
          colbase = lax.bitwise_and(iota + (d8 * 8 + dd),
                                    jnp.full((L,), L - 1, jnp.int32))
          for c0 in range(0, FEATURES, L):
            cvec = colbase + c0
            for g in range(8):
              ilvec = g * L + iota
              v = plsc.load_gather(vbuf[slot], [cvec, ilvec])
              plsc.store_scatter(obuf[slot], [base_g[g] + cvec], v)
        return carry

      lax.fori_loop(0, 2, dbody, jnp.int32(0))

    def work(t, slot):
      pltpu.make_async_copy(tab_hbm.at[:, pl.ds(wtile(t) * 128, 128)],
                            vbuf[slot], isem[slot]).wait()
      transpose_block(slot)
      pltpu.async_copy(obuf[slot], out_hbm.at[pl.ds(wtile(t) * 4096, 4096)],
                       osem[slot])

    nt = TPW + jnp.where(wid < 4, 1, 0)

    for s in range(RBUF):
      stage(jnp.int32(s), s)

    def body(j, carry):
      for s in range(RBUF):
        t = j * RBUF + s

        @pl.when(t < nt)
        def _():
          @pl.when(t >= RBUF)
          def _():
            pltpu.make_async_copy(
                obuf[s], out_hbm.at[pl.ds(wtile(t) * 4096, 4096)],
                osem[s]).wait()
          work(t, s)

          @pl.when(t + RBUF < nt)
          def _():
            stage(t + RBUF, s)
      return carry

    lax.fori_loop(0, (TPW + 1 + RBUF - 1) // RBUF, body, jnp.int32(0))
    for s in range(RBUF):
      @pl.when(jnp.minimum(nt, jnp.int32(RBUF)) > s)
      def _():
        pltpu.make_async_copy(obuf[s], out_hbm.at[pl.ds(0, 4096)],
                              osem[s]).wait()

    @pl.when(wid == 4)
    def _():
      pltpu.sync_copy(tail_hbm, vbuf[0])
      transpose_block(0)
      pltpu.sync_copy(obuf[0].at[pl.ds(0, TAIL * FEATURES)],
                      out_hbm.at[pl.ds(NTILE * 4096, TAIL * FEATURES)])

  return k


def _build_gather():
  @functools.partial(
      pl.kernel,
      mesh=_mesh,
      compiler_params=pltpu.CompilerParams(needs_layout_passes=False),
      out_type=jax.ShapeDtypeStruct((HIST, FEATURES, BATCH), jnp.float32),
      scratch_types=[
          pltpu.VMEM((HIST, BBLK), jnp.int32),
          [pltpu.VMEM((BBLK,), jnp.int32) for _ in range(NBUF)],
          [pltpu.VMEM((BBLK,), jnp.int32) for _ in range(NBUF)],
          [pltpu.VMEM((BBLK, 128), jnp.float32) for _ in range(NBUF)],
          [pltpu.VMEM((FEATURES, BBLK), jnp.float32) for _ in range(NBUF)],
          [pltpu.SemaphoreType.DMA for _ in range(NBUF)],
          [pltpu.SemaphoreType.DMA for _ in range(NBUF)],
      ],
  )
  def k(idx_hbm, table_hbm, out_hbm, idx_v, rowid_v, qoff_v, buf_v, obuf_v,
        gsem, osem):
    wid = lax.axis_index("s") * 2 + lax.axis_index("c")
    b0 = wid * BBLK
    pltpu.sync_copy(idx_hbm.at[:, pl.ds(b0, BBLK)], idx_v)

    iota = lax.iota(jnp.int32, L)

    def prep(h, slot):
      for g in range(BBLK // L):
        v = idx_v[h, pl.ds(g * L, L)]
        rowid_v[slot][pl.ds(g * L, L)] = v >> 2
        qoff_v[slot][pl.ds(g * L, L)] = (v & 3) << 5
      pltpu.async_copy(table_hbm.at[rowid_v[slot]], buf_v[slot], gsem[slot])

    def step(h, slot):
      pltpu.make_async_copy(
          table_hbm.at[rowid_v[slot]], buf_v[slot], gsem[slot]).wait()

      def dbody(d, carry):
        colbase = lax.bitwise_and(iota + d, jnp.full((L,), L - 1, jnp.int32))
        for g in range(BBLK // L):
          rows = g * L + iota
          qoff = qoff_v[slot][pl.ds(g * L, L)]
          for c0 in range(0, FEATURES, L):
            cols = colbase + c0
            v = plsc.load_gather(buf_v[slot], [rows, qoff + cols])
            plsc.store_scatter(obuf_v[slot], [cols, rows], v)
        return carry

      lax.fori_loop(0, L, dbody, jnp.int32(0))
      pltpu.async_copy(obuf_v[slot], out_hbm.at[h, :, pl.ds(b0, BBLK)],
                       osem[slot])
      prep(jnp.minimum(h + NBUF, HIST - 1), slot)

    for s in range(NBUF):
      prep(jnp.int32(s), s)

    def body(j, carry):
      for s in range(NBUF):
        h = j * NBUF + s

        @pl.when(h >= NBUF)
        def _():
          pltpu.make_async_copy(
              obuf_v[s], out_hbm.at[h, :, pl.ds(b0, BBLK)], osem[s]).wait()
        step(h, s)
      return carry

    lax.fori_loop(0, HIST // NBUF, body, jnp.int32(0))

    for s in range(NBUF):
      pltpu.make_async_copy(
          table_hbm.at[rowid_v[s]], buf_v[s], gsem[s]).wait()
      pltpu.make_async_copy(
          obuf_v[s], out_hbm.at[jnp.int32(0), :, pl.ds(b0, BBLK)],
          osem[s]).wait()

  return k


_repack_kernel = _build_repack()
_gather_kernel = _build_gather()


def kernel(inputs, embedding):
  idx_t = inputs.T.astype(jnp.int32)
  tail = jnp.pad(embedding[NTILE * 128:].T, ((0, 0), (0, 128 - TAIL)))
  packed = _repack_kernel(embedding.T, tail)
  table128 = packed.reshape(QROWS, 128)
  out_t = _gather_kernel(idx_t, table128)
  return out_t.transpose(2, 0, 1)
